# baseline (speedup 1.0000x reference)
import functools

import jax
import jax.numpy as jnp
from jax import lax
from jax.experimental import pallas as pl
from jax.experimental.pallas import tpu as pltpu
from jax.experimental.pallas import tpu_sc as plsc

N_NODES = 10000
N_EDGES = 320000
D = 128

NC = 2
NS = 16
NW = NC * NS
EPT = N_EDGES // NW
K = 80
NB = EPT // K
SB = 25
NSUP = NB // SB
RB = 624
TAIL = N_NODES - NS * RB


def _mm_body(x_ref, wt_ref, b_ref, o_ref):
    o_ref[...] = (
        jnp.dot(x_ref[...], wt_ref[...], preferred_element_type=jnp.float32)
        + b_ref[...]
    )


def _matmul(x, wt, b2):
    blk = 1000
    return pl.pallas_call(
        _mm_body,
        grid=(N_NODES // blk,),
        in_specs=[
            pl.BlockSpec((blk, D), lambda i: (i, 0)),
            pl.BlockSpec((D, D), lambda i: (0, 0)),
            pl.BlockSpec((1, D), lambda i: (0, 0)),
        ],
        out_specs=pl.BlockSpec((blk, D), lambda i: (i, 0)),
        out_shape=jax.ShapeDtypeStruct((N_NODES, D), jnp.float32),
    )(x, wt, b2)


_mesh = plsc.VectorSubcoreMesh(core_axis_name="c", subcore_axis_name="s")


@functools.partial(
    pl.kernel,
    mesh=_mesh,
    out_type=jax.ShapeDtypeStruct((NC * N_NODES, D), jnp.float32),
    scratch_types=[
        pltpu.VMEM((SB, K), jnp.int32),
        pltpu.VMEM((SB, K), jnp.int32),
        pltpu.VMEM((SB, K), jnp.float32),
        pltpu.VMEM((3 * K, D), jnp.float32),
        pltpu.VMEM_SHARED((N_NODES, D), jnp.float32),
        pltpu.SemaphoreType.DMA,
        pltpu.SemaphoreType.DMA,
        pltpu.SemaphoreType.DMA,
        pltpu.SemaphoreType.DMA,
        pltpu.SemaphoreType.DMA,
    ],
)
def _sc_edges(out_hbm, col_hbm, row_hbm, val_hbm, part_hbm,
              colv, rowv, valv, rbuf, agg, gsem, gsem2, gsem3, gsem4, ssem):
    c = lax.axis_index("c")
    s = lax.axis_index("s")
    wid = c * NS + s

    def _zrow(i, carry):
        for j in range(D // 16):
            rbuf[i, pl.ds(j * 16, 16)] = jnp.zeros((16,), jnp.float32)
        return carry

    lax.fori_loop(0, K, _zrow, 0)
    zsrc = rbuf.at[pl.ds(0, K)]
    for kk in range(RB // K):
        pltpu.sync_copy(zsrc, agg.at[pl.ds(s * RB + kk * K, K)])
    rem = RB - (RB // K) * K
    if rem:
        pltpu.sync_copy(
            rbuf.at[pl.ds(0, rem)],
            agg.at[pl.ds(s * RB + (RB // K) * K, rem)],
        )

    @pl.when(s == NS - 1)
    def _ztail():
        pltpu.sync_copy(rbuf.at[pl.ds(0, TAIL)], agg.at[pl.ds(NS * RB, TAIL)])

    plsc.subcore_barrier()

    H = K // 2
    Q = K // 4
    gsems = (gsem, gsem2, gsem3, gsem4)

    def _super(u, carry):
        pltpu.sync_copy(col_hbm.at[wid, u], colv)
        pltpu.sync_copy(row_hbm.at[wid, u], rowv)
        pltpu.sync_copy(val_hbm.at[wid, u], valv)

        for hh in range(4):
            pltpu.async_copy(
                out_hbm.at[colv.at[0, pl.ds(hh * Q, Q)]],
                rbuf.at[pl.ds(hh * Q, Q)],
                gsems[hh],
            )

        def _block(b, c1):
            off = lax.rem(b, 3) * K
            qoff = lax.rem(b + 1, 3) * K

            for hh in range(4):
                pltpu.make_async_copy(
                    out_hbm.at[colv.at[b, pl.ds(hh * Q, Q)]],
                    rbuf.at[pl.ds(off + hh * Q, Q)],
                    gsems[hh],
                ).wait()

            @pl.when(b + 1 < SB)
            def _issue_next():
                @pl.when(b >= 2)
                def _drain_prev_scatter():
                    pltpu.make_async_copy(
                        rbuf.at[pl.ds(qoff, K)],
                        agg.at[rowv.at[b - 2]],
                        ssem,
                    ).wait()

                for hh in range(4):
                    pltpu.async_copy(
                        out_hbm.at[colv.at[b + 1, pl.ds(hh * Q, Q)]],
                        rbuf.at[pl.ds(qoff + hh * Q, Q)],
                        gsems[hh],
                    )

            def _grp(t, c2):
                vchunk = valv[b, pl.ds(t * 16, 16)]
                for e16 in range(16):
                    v = vchunk[e16]
                    e = off + t * 16 + e16
                    for j in range(D // 16):
                        sl = pl.ds(j * 16, 16)
                        rbuf[e, sl] = rbuf[e, sl] * v
                return c2

            lax.fori_loop(0, K // 16, _grp, 0)

            pltpu.async_copy(
                rbuf.at[pl.ds(off, K)], agg.at[rowv.at[b]], ssem, add=True
            )
            return c1

        lax.fori_loop(0, SB, _block, 0)

        for bb in (SB - 3, SB - 2, SB - 1):
            pltpu.make_async_copy(
                rbuf.at[pl.ds((bb % 3) * K, K)], agg.at[rowv.at[bb]], ssem
            ).wait()
        return carry

    lax.fori_loop(0, NSUP, _super, 0)

    plsc.subcore_barrier()
    pltpu.sync_copy(
        agg.at[pl.ds(s * RB, RB)],
        part_hbm.at[pl.ds(c * N_NODES + s * RB, RB)],
    )

    @pl.when(s == NS - 1)
    def _wtail():
        pltpu.sync_copy(
            agg.at[pl.ds(NS * RB, TAIL)],
            part_hbm.at[pl.ds(c * N_NODES + NS * RB, TAIL)],
        )


def _combine_body(p0_ref, p1_ref, o_ref):
    a = p0_ref[...] + p1_ref[...]
    o_ref[...] = jnp.where(a > 0.0, a, jnp.exp(a) - 1.0)


def _combine(parts):
    blk = 1000
    nblk = N_NODES // blk
    return pl.pallas_call(
        _combine_body,
        grid=(nblk,),
        in_specs=[
            pl.BlockSpec((blk, D), lambda i: (i, 0)),
            pl.BlockSpec((blk, D), lambda i: (nblk + i, 0)),
        ],
        out_specs=pl.BlockSpec((blk, D), lambda i: (i, 0)),
        out_shape=jax.ShapeDtypeStruct((N_NODES, D), jnp.float32),
    )(parts, parts)


def kernel(x, adj_indices, adj_values, W, b):
    row = adj_indices[0].astype(jnp.int32).reshape(NW, NSUP, SB, K)
    col = adj_indices[1].astype(jnp.int32).reshape(NW, NSUP, SB, K)
    vals = adj_values.astype(jnp.float32).reshape(NW, NSUP, SB, K)
    out = _matmul(x, W.T, b.reshape(1, D))
    parts = _sc_edges(out, col, row, vals)
    return _combine(parts)


# speedup vs baseline: 1.1475x; 1.1475x over previous
import functools

import jax
import jax.numpy as jnp
from jax import lax
from jax.experimental import pallas as pl
from jax.experimental.pallas import tpu as pltpu
from jax.experimental.pallas import tpu_sc as plsc

N_NODES = 10000
N_EDGES = 320000
D = 128

NC = 2
NS = 16
NW = NC * NS
EPT = N_EDGES // NW
K = 80
NB = EPT // K
SB = 25
NSUP = NB // SB
RB = 624
TAIL = N_NODES - NS * RB


def _mm_body(x_ref, wt_ref, b_ref, o_ref):
    o_ref[...] = (
        jnp.dot(x_ref[...], wt_ref[...], preferred_element_type=jnp.float32)
        + b_ref[...]
    )


def _matmul(x, wt, b2):
    blk = 1000
    return pl.pallas_call(
        _mm_body,
        grid=(N_NODES // blk,),
        in_specs=[
            pl.BlockSpec((blk, D), lambda i: (i, 0)),
            pl.BlockSpec((D, D), lambda i: (0, 0)),
            pl.BlockSpec((1, D), lambda i: (0, 0)),
        ],
        out_specs=pl.BlockSpec((blk, D), lambda i: (i, 0)),
        out_shape=jax.ShapeDtypeStruct((N_NODES, D), jnp.float32),
    )(x, wt, b2)


_mesh = plsc.VectorSubcoreMesh(core_axis_name="c", subcore_axis_name="s")


@functools.partial(
    pl.kernel,
    mesh=_mesh,
    out_type=jax.ShapeDtypeStruct((NC * N_NODES, D), jnp.float32),
    scratch_types=[
        pltpu.VMEM((SB, K), jnp.int32),
        pltpu.VMEM((SB, K), jnp.int32),
        pltpu.VMEM((SB, K), jnp.float32),
        pltpu.VMEM((3 * K, D), jnp.float32),
        pltpu.VMEM_SHARED((N_NODES, D), jnp.float32),
        pltpu.SemaphoreType.DMA,
        pltpu.SemaphoreType.DMA,
        pltpu.SemaphoreType.DMA,
    ],
)
def _sc_edges(out_hbm, col_hbm, row_hbm, val_hbm, part_hbm,
              colv, rowv, valv, rbuf, agg, gsem, gsem2, ssem):
    c = lax.axis_index("c")
    s = lax.axis_index("s")
    wid = c * NS + s

    def _zrow(i, carry):
        for j in range(D // 16):
            rbuf[i, pl.ds(j * 16, 16)] = jnp.zeros((16,), jnp.float32)
        return carry

    lax.fori_loop(0, K, _zrow, 0)
    zsrc = rbuf.at[pl.ds(0, K)]
    for kk in range(RB // K):
        pltpu.sync_copy(zsrc, agg.at[pl.ds(s * RB + kk * K, K)])
    rem = RB - (RB // K) * K
    if rem:
        pltpu.sync_copy(
            rbuf.at[pl.ds(0, rem)],
            agg.at[pl.ds(s * RB + (RB // K) * K, rem)],
        )

    @pl.when(s == NS - 1)
    def _ztail():
        pltpu.sync_copy(rbuf.at[pl.ds(0, TAIL)], agg.at[pl.ds(NS * RB, TAIL)])

    plsc.subcore_barrier()

    H = K // 2

    def _super(u, carry):
        pltpu.sync_copy(col_hbm.at[wid, u], colv)
        pltpu.sync_copy(row_hbm.at[wid, u], rowv)
        pltpu.sync_copy(val_hbm.at[wid, u], valv)

        pltpu.async_copy(
            out_hbm.at[colv.at[0, pl.ds(0, H)]], rbuf.at[pl.ds(0, H)], gsem
        )
        pltpu.async_copy(
            out_hbm.at[colv.at[0, pl.ds(H, H)]], rbuf.at[pl.ds(H, H)], gsem2
        )

        def _block(b, c1):
            off = lax.rem(b, 3) * K
            qoff = lax.rem(b + 1, 3) * K

            @pl.when(b + 1 < SB)
            def _issue_next():
                @pl.when(b >= 2)
                def _drain_prev_scatter():
                    pltpu.make_async_copy(
                        rbuf.at[pl.ds(qoff, K)],
                        agg.at[rowv.at[b - 2]],
                        ssem,
                    ).wait()

                pltpu.async_copy(
                    out_hbm.at[colv.at[b + 1, pl.ds(0, H)]],
                    rbuf.at[pl.ds(qoff, H)], gsem
                )
                pltpu.async_copy(
                    out_hbm.at[colv.at[b + 1, pl.ds(H, H)]],
                    rbuf.at[pl.ds(qoff + H, H)], gsem2
                )

            pltpu.make_async_copy(
                out_hbm.at[colv.at[b, pl.ds(0, H)]],
                rbuf.at[pl.ds(off, H)], gsem
            ).wait()
            pltpu.make_async_copy(
                out_hbm.at[colv.at[b, pl.ds(H, H)]],
                rbuf.at[pl.ds(off + H, H)], gsem2
            ).wait()

            def _grp(t, c2):
                vchunk = valv[b, pl.ds(t * 16, 16)]
                for e16 in range(16):
                    v = vchunk[e16]
                    e = off + t * 16 + e16
                    for j in range(D // 16):
                        sl = pl.ds(j * 16, 16)
                        rbuf[e, sl] = rbuf[e, sl] * v
                return c2

            lax.fori_loop(0, K // 16, _grp, 0)

            pltpu.async_copy(
                rbuf.at[pl.ds(off, K)], agg.at[rowv.at[b]], ssem, add=True
            )
            return c1

        lax.fori_loop(0, SB, _block, 0)

        for bb in (SB - 3, SB - 2, SB - 1):
            pltpu.make_async_copy(
                rbuf.at[pl.ds((bb % 3) * K, K)], agg.at[rowv.at[bb]], ssem
            ).wait()
        return carry

    lax.fori_loop(0, NSUP, _super, 0)

    plsc.subcore_barrier()
    pltpu.sync_copy(
        agg.at[pl.ds(s * RB, RB)],
        part_hbm.at[pl.ds(c * N_NODES + s * RB, RB)],
    )

    @pl.when(s == NS - 1)
    def _wtail():
        pltpu.sync_copy(
            agg.at[pl.ds(NS * RB, TAIL)],
            part_hbm.at[pl.ds(c * N_NODES + NS * RB, TAIL)],
        )


def _combine_body(p0_ref, p1_ref, o_ref):
    a = p0_ref[...] + p1_ref[...]
    o_ref[...] = jnp.where(a > 0.0, a, jnp.exp(a) - 1.0)


def _combine(parts):
    blk = 1000
    nblk = N_NODES // blk
    return pl.pallas_call(
        _combine_body,
        grid=(nblk,),
        in_specs=[
            pl.BlockSpec((blk, D), lambda i: (i, 0)),
            pl.BlockSpec((blk, D), lambda i: (nblk + i, 0)),
        ],
        out_specs=pl.BlockSpec((blk, D), lambda i: (i, 0)),
        out_shape=jax.ShapeDtypeStruct((N_NODES, D), jnp.float32),
    )(parts, parts)


def kernel(x, adj_indices, adj_values, W, b):
    row = adj_indices[0].astype(jnp.int32).reshape(NW, NSUP, SB, K)
    col = adj_indices[1].astype(jnp.int32).reshape(NW, NSUP, SB, K)
    vals = adj_values.astype(jnp.float32).reshape(NW, NSUP, SB, K)
    out = _matmul(x, W.T, b.reshape(1, D))
    parts = _sc_edges(out, col, row, vals)
    return _combine(parts)
